# baseline (speedup 1.0000x reference)
import numpy as np
import jax
import jax.numpy as jnp
from jax.experimental import pallas as pl
from jax.experimental.pallas import tpu as pltpu


_TILE_N = 8192

_PERM_T = np.zeros((30, 30), np.float32)
for _o in range(10):
    for _l in range(3):
        _PERM_T[_o * 3 + _l, _l * 10 + _o] = 1.0


def _fused_kernel(x0_ref, x1_ref, x2_ref,
                  w0_ref, w1_ref, w2_ref,
                  b0_ref, b1_ref, b2_ref,
                  cw_ref, cb_ref, perm_ref,
                  o_ref):
    f32 = jnp.float32
    cw = cw_ref[...]
    g = [jnp.transpose(cw[:, k, :]) for k in range(3)]
    z = jnp.zeros_like(g[0])
    wc = [
        jnp.concatenate([g[1], g[0], z], axis=0),
        jnp.concatenate([g[2], g[1], g[0]], axis=0),
        jnp.concatenate([z, g[2], g[1]], axis=0),
    ]
    dn_bt = (((1,), (1,)), ((), ()))
    m0 = jax.lax.dot_general(wc[0], w0_ref[...], dn_bt, preferred_element_type=f32)
    m1 = jax.lax.dot_general(wc[1], w1_ref[...], dn_bt, preferred_element_type=f32)
    m2 = jax.lax.dot_general(wc[2], w2_ref[...], dn_bt, preferred_element_type=f32)
    cb_col = jnp.transpose(cb_ref[...])
    beta = (jnp.concatenate([cb_col, cb_col, cb_col], axis=0)
            + jax.lax.dot_general(wc[0], b0_ref[...], dn_bt, preferred_element_type=f32)
            + jax.lax.dot_general(wc[1], b1_ref[...], dn_bt, preferred_element_type=f32)
            + jax.lax.dot_general(wc[2], b2_ref[...], dn_bt, preferred_element_type=f32))
    acc = jnp.dot(m0, x0_ref[...], preferred_element_type=f32)
    acc = acc + jnp.dot(m1, x1_ref[...], preferred_element_type=f32)
    acc = acc + jnp.dot(m2, x2_ref[...], preferred_element_type=f32)
    o_ref[...] = jnp.dot(perm_ref[...], acc + beta,
                         preferred_element_type=f32).astype(o_ref.dtype)


def kernel(x_maccs, x_estate, x_attrmask,
           linear_w_0, linear_w_1, linear_w_2,
           linear_b_0, linear_b_1, linear_b_2,
           conv_w, conv_b):
    B = x_maccs.shape[0]
    D0 = x_maccs.shape[1]
    D1 = x_estate.shape[1]
    D2 = x_attrmask.shape[1]
    C = linear_w_0.shape[0]
    O = conv_w.shape[0]
    N = O * 3

    f32 = jnp.float32
    x0t = jnp.transpose(x_maccs.astype(f32))
    x1t = jnp.transpose(x_estate.astype(f32))
    x2t = jnp.transpose(x_attrmask.astype(f32))
    w0t = jnp.transpose(linear_w_0.astype(f32))
    w1t = jnp.transpose(linear_w_1.astype(f32))
    w2t = jnp.transpose(linear_w_2.astype(f32))
    cwt = jnp.transpose(conv_w.astype(f32), (1, 2, 0))
    b0 = linear_b_0.astype(f32).reshape(1, C)
    b1 = linear_b_1.astype(f32).reshape(1, C)
    b2 = linear_b_2.astype(f32).reshape(1, C)
    cb = conv_b.astype(f32).reshape(1, O)
    perm = jnp.asarray(_PERM_T)

    tn = min(_TILE_N, B)
    grid = pl.cdiv(B, tn)

    out_t = pl.pallas_call(
        _fused_kernel,
        out_shape=jax.ShapeDtypeStruct((N, B), f32),
        grid_spec=pltpu.PrefetchScalarGridSpec(
            num_scalar_prefetch=0,
            grid=(grid,),
            in_specs=[
                pl.BlockSpec((D0, tn), lambda i: (0, i)),
                pl.BlockSpec((D1, tn), lambda i: (0, i)),
                pl.BlockSpec((D2, tn), lambda i: (0, i)),
                pl.BlockSpec((D0, C), lambda i: (0, 0)),
                pl.BlockSpec((D1, C), lambda i: (0, 0)),
                pl.BlockSpec((D2, C), lambda i: (0, 0)),
                pl.BlockSpec((1, C), lambda i: (0, 0)),
                pl.BlockSpec((1, C), lambda i: (0, 0)),
                pl.BlockSpec((1, C), lambda i: (0, 0)),
                pl.BlockSpec((C, 3, O), lambda i: (0, 0, 0)),
                pl.BlockSpec((1, O), lambda i: (0, 0)),
                pl.BlockSpec((N, N), lambda i: (0, 0)),
            ],
            out_specs=pl.BlockSpec((N, tn), lambda i: (0, i)),
        ),
        compiler_params=pltpu.CompilerParams(
            dimension_semantics=("parallel",)),
    )(x0t, x1t, x2t, w0t, w1t, w2t, b0, b1, b2, cwt, cb, perm)
    return jnp.transpose(out_t)


# speedup vs baseline: 1.0447x; 1.0447x over previous
import numpy as np
import jax
import jax.numpy as jnp
from jax.experimental import pallas as pl
from jax.experimental.pallas import tpu as pltpu


_TILE_N = 4096

_PERM_T = np.zeros((30, 30), np.float32)
for _o in range(10):
    for _l in range(3):
        _PERM_T[_o * 3 + _l, _l * 10 + _o] = 1.0


def _fused_kernel(x0_ref, x1_ref, x2_ref,
                  w0_ref, w1_ref, w2_ref,
                  b0_ref, b1_ref, b2_ref,
                  cw_ref, cb_ref, perm_ref,
                  o_ref,
                  m0_ref, m1_ref, m2_ref, beta_ref):
    f32 = jnp.float32

    @pl.when(pl.program_id(0) == 0)
    def _fold():
        cw = cw_ref[...]
        g = [jnp.transpose(cw[:, k, :]) for k in range(3)]
        z = jnp.zeros_like(g[0])
        wc = [
            jnp.concatenate([g[1], g[0], z], axis=0),
            jnp.concatenate([g[2], g[1], g[0]], axis=0),
            jnp.concatenate([z, g[2], g[1]], axis=0),
        ]
        dn_bt = (((1,), (1,)), ((), ()))
        perm = perm_ref[...]
        m0 = jax.lax.dot_general(wc[0], w0_ref[...], dn_bt, preferred_element_type=f32)
        m1 = jax.lax.dot_general(wc[1], w1_ref[...], dn_bt, preferred_element_type=f32)
        m2 = jax.lax.dot_general(wc[2], w2_ref[...], dn_bt, preferred_element_type=f32)
        m0_ref[...] = jnp.dot(perm, m0, preferred_element_type=f32)
        m1_ref[...] = jnp.dot(perm, m1, preferred_element_type=f32)
        m2_ref[...] = jnp.dot(perm, m2, preferred_element_type=f32)
        cb_col = jnp.transpose(cb_ref[...])
        beta = (jnp.concatenate([cb_col, cb_col, cb_col], axis=0)
                + jax.lax.dot_general(wc[0], b0_ref[...], dn_bt, preferred_element_type=f32)
                + jax.lax.dot_general(wc[1], b1_ref[...], dn_bt, preferred_element_type=f32)
                + jax.lax.dot_general(wc[2], b2_ref[...], dn_bt, preferred_element_type=f32))
        beta_ref[...] = jnp.dot(perm, beta, preferred_element_type=f32)

    acc = jnp.dot(m0_ref[...], x0_ref[...], preferred_element_type=f32)
    acc = acc + jnp.dot(m1_ref[...], x1_ref[...], preferred_element_type=f32)
    acc = acc + jnp.dot(m2_ref[...], x2_ref[...], preferred_element_type=f32)
    o_ref[...] = (acc + beta_ref[...]).astype(o_ref.dtype)


def kernel(x_maccs, x_estate, x_attrmask,
           linear_w_0, linear_w_1, linear_w_2,
           linear_b_0, linear_b_1, linear_b_2,
           conv_w, conv_b):
    B = x_maccs.shape[0]
    D0 = x_maccs.shape[1]
    D1 = x_estate.shape[1]
    D2 = x_attrmask.shape[1]
    C = linear_w_0.shape[0]
    O = conv_w.shape[0]
    N = O * 3

    f32 = jnp.float32
    x0t = jnp.transpose(x_maccs.astype(f32))
    x1t = jnp.transpose(x_estate.astype(f32))
    x2t = jnp.transpose(x_attrmask.astype(f32))
    w0t = jnp.transpose(linear_w_0.astype(f32))
    w1t = jnp.transpose(linear_w_1.astype(f32))
    w2t = jnp.transpose(linear_w_2.astype(f32))
    cwt = jnp.transpose(conv_w.astype(f32), (1, 2, 0))
    b0 = linear_b_0.astype(f32).reshape(1, C)
    b1 = linear_b_1.astype(f32).reshape(1, C)
    b2 = linear_b_2.astype(f32).reshape(1, C)
    cb = conv_b.astype(f32).reshape(1, O)
    perm = jnp.asarray(_PERM_T)

    tn = min(_TILE_N, B)
    grid = pl.cdiv(B, tn)

    out_t = pl.pallas_call(
        _fused_kernel,
        out_shape=jax.ShapeDtypeStruct((N, B), f32),
        grid_spec=pltpu.PrefetchScalarGridSpec(
            num_scalar_prefetch=0,
            grid=(grid,),
            in_specs=[
                pl.BlockSpec((D0, tn), lambda i: (0, i)),
                pl.BlockSpec((D1, tn), lambda i: (0, i)),
                pl.BlockSpec((D2, tn), lambda i: (0, i)),
                pl.BlockSpec((D0, C), lambda i: (0, 0)),
                pl.BlockSpec((D1, C), lambda i: (0, 0)),
                pl.BlockSpec((D2, C), lambda i: (0, 0)),
                pl.BlockSpec((1, C), lambda i: (0, 0)),
                pl.BlockSpec((1, C), lambda i: (0, 0)),
                pl.BlockSpec((1, C), lambda i: (0, 0)),
                pl.BlockSpec((C, 3, O), lambda i: (0, 0, 0)),
                pl.BlockSpec((1, O), lambda i: (0, 0)),
                pl.BlockSpec((N, N), lambda i: (0, 0)),
            ],
            out_specs=pl.BlockSpec((N, tn), lambda i: (0, i)),
            scratch_shapes=[
                pltpu.VMEM((N, D0), f32),
                pltpu.VMEM((N, D1), f32),
                pltpu.VMEM((N, D2), f32),
                pltpu.VMEM((N, 1), f32),
            ],
        ),
        compiler_params=pltpu.CompilerParams(
            dimension_semantics=("arbitrary",)),
    )(x0t, x1t, x2t, w0t, w1t, w2t, b0, b1, b2, cwt, cb, perm)
    return jnp.transpose(out_t)


# speedup vs baseline: 1.1793x; 1.1288x over previous
import numpy as np
import jax
import jax.numpy as jnp
from jax.experimental import pallas as pl
from jax.experimental.pallas import tpu as pltpu


_TILE_N = 4096

_PERM_T = np.zeros((30, 30), np.float32)
for _o in range(10):
    for _l in range(3):
        _PERM_T[_o * 3 + _l, _l * 10 + _o] = 1.0


def _fused_kernel(x0_ref, x1_ref, x2_ref,
                  w0_ref, w1_ref, w2_ref,
                  b0_ref, b1_ref, b2_ref,
                  cw_ref, cb_ref, perm_ref,
                  o_ref,
                  m0_ref, m1_ref, m2_ref, beta_ref):
    f32 = jnp.float32

    @pl.when(pl.program_id(0) == 0)
    def _fold():
        cw = cw_ref[...]
        g = [cw[:, k, :] for k in range(3)]
        z = jnp.zeros_like(g[0])
        wc = [
            jnp.concatenate([g[1], g[0], z], axis=0),
            jnp.concatenate([g[2], g[1], g[0]], axis=0),
            jnp.concatenate([z, g[2], g[1]], axis=0),
        ]
        dn_bt = (((1,), (1,)), ((), ()))
        perm = perm_ref[...]
        m0 = jax.lax.dot_general(wc[0], w0_ref[...], dn_bt, preferred_element_type=f32)
        m1 = jax.lax.dot_general(wc[1], w1_ref[...], dn_bt, preferred_element_type=f32)
        m2 = jax.lax.dot_general(wc[2], w2_ref[...], dn_bt, preferred_element_type=f32)
        m0_ref[...] = jnp.dot(perm, m0, preferred_element_type=f32)
        m1_ref[...] = jnp.dot(perm, m1, preferred_element_type=f32)
        m2_ref[...] = jnp.dot(perm, m2, preferred_element_type=f32)
        cb_col = jnp.transpose(cb_ref[...])
        beta = (jnp.concatenate([cb_col, cb_col, cb_col], axis=0)
                + jax.lax.dot_general(wc[0], b0_ref[...], dn_bt, preferred_element_type=f32)
                + jax.lax.dot_general(wc[1], b1_ref[...], dn_bt, preferred_element_type=f32)
                + jax.lax.dot_general(wc[2], b2_ref[...], dn_bt, preferred_element_type=f32))
        beta_ref[...] = jnp.dot(perm, beta, preferred_element_type=f32)

    acc = jnp.dot(m0_ref[...], x0_ref[...], preferred_element_type=f32)
    acc = acc + jnp.dot(m1_ref[...], x1_ref[...], preferred_element_type=f32)
    acc = acc + jnp.dot(m2_ref[...], x2_ref[...], preferred_element_type=f32)
    o_ref[...] = (acc + beta_ref[...]).astype(o_ref.dtype)


def kernel(x_maccs, x_estate, x_attrmask,
           linear_w_0, linear_w_1, linear_w_2,
           linear_b_0, linear_b_1, linear_b_2,
           conv_w, conv_b):
    B = x_maccs.shape[0]
    D0 = x_maccs.shape[1]
    D1 = x_estate.shape[1]
    D2 = x_attrmask.shape[1]
    C = linear_w_0.shape[0]
    O = conv_w.shape[0]
    N = O * 3

    f32 = jnp.float32
    x0t = jnp.transpose(x_maccs.astype(f32))
    x1t = jnp.transpose(x_estate.astype(f32))
    x2t = jnp.transpose(x_attrmask.astype(f32))
    w0t = jnp.transpose(linear_w_0.astype(f32))
    w1t = jnp.transpose(linear_w_1.astype(f32))
    w2t = jnp.transpose(linear_w_2.astype(f32))
    cwt = jnp.transpose(conv_w.astype(f32), (0, 2, 1))
    b0 = linear_b_0.astype(f32).reshape(1, C)
    b1 = linear_b_1.astype(f32).reshape(1, C)
    b2 = linear_b_2.astype(f32).reshape(1, C)
    cb = conv_b.astype(f32).reshape(1, O)
    perm = jnp.asarray(_PERM_T)

    tn = min(_TILE_N, B)
    grid = pl.cdiv(B, tn)

    out_t = pl.pallas_call(
        _fused_kernel,
        out_shape=jax.ShapeDtypeStruct((N, B), f32),
        grid_spec=pltpu.PrefetchScalarGridSpec(
            num_scalar_prefetch=0,
            grid=(grid,),
            in_specs=[
                pl.BlockSpec((D0, tn), lambda i: (0, i)),
                pl.BlockSpec((D1, tn), lambda i: (0, i)),
                pl.BlockSpec((D2, tn), lambda i: (0, i)),
                pl.BlockSpec((D0, C), lambda i: (0, 0)),
                pl.BlockSpec((D1, C), lambda i: (0, 0)),
                pl.BlockSpec((D2, C), lambda i: (0, 0)),
                pl.BlockSpec((1, C), lambda i: (0, 0)),
                pl.BlockSpec((1, C), lambda i: (0, 0)),
                pl.BlockSpec((1, C), lambda i: (0, 0)),
                pl.BlockSpec((O, 3, C), lambda i: (0, 0, 0)),
                pl.BlockSpec((1, O), lambda i: (0, 0)),
                pl.BlockSpec((N, N), lambda i: (0, 0)),
            ],
            out_specs=pl.BlockSpec((N, tn), lambda i: (0, i)),
            scratch_shapes=[
                pltpu.VMEM((N, D0), f32),
                pltpu.VMEM((N, D1), f32),
                pltpu.VMEM((N, D2), f32),
                pltpu.VMEM((N, 1), f32),
            ],
        ),
        compiler_params=pltpu.CompilerParams(
            dimension_semantics=("arbitrary",)),
    )(x0t, x1t, x2t, w0t, w1t, w2t, b0, b1, b2, cwt, cb, perm)
    return jnp.transpose(out_t)
